# baseline (speedup 1.0000x reference)
import functools

import jax
import jax.numpy as jnp
from jax import lax
from jax.experimental import pallas as pl
from jax.experimental.pallas import tpu as pltpu
from jax.experimental.pallas import tpu_sc as plsc

_NC = 2
_NS = 16
_NW = _NC * _NS
_CHUNK = 128
_R = 1000


def _ceil_div(a, b):
    return -(-a // b)


def _stripe_rows(n_nodes):
    return _ceil_div(n_nodes + 1, _NS * _CHUNK) * _CHUNK


_NSEG = 2


@functools.lru_cache(maxsize=None)
def _make_edge_pass(n_nodes, d, nchunks):
    rstripe = _stripe_rows(n_nodes)
    acc_rows = rstripe * _NS
    seg = nchunks // _NSEG

    def body(y_hbm, src_hbm, dst_hbm, out_hbm, src_v, dst_v,
             rows_a, rows_b, acc, sem_a, sem_b):
        c = lax.axis_index("c")
        s = lax.axis_index("s")
        wid = s * _NC + c

        def zero_body(i, carry):
            rows_a[i // (d // 16), pl.ds((i % (d // 16)) * 16, 16)] = (
                jnp.zeros((16,), jnp.float32))
            return carry
        lax.fori_loop(0, _CHUNK * (d // 16), zero_body, 0)

        base = s * rstripe
        nfull = rstripe // _CHUNK
        rem = rstripe - nfull * _CHUNK
        for k in range(nfull):
            pltpu.sync_copy(rows_a, acc.at[pl.ds(base + k * _CHUNK, _CHUNK)])
        if rem:
            pltpu.sync_copy(rows_a.at[pl.ds(0, rem)],
                            acc.at[pl.ds(base + nfull * _CHUNK, rem)])

        plsc.subcore_barrier()

        for ph in range(_NSEG):
            off = ph * seg
            pltpu.sync_copy(src_hbm.at[wid].at[pl.ds(off, seg)], src_v)
            pltpu.sync_copy(dst_hbm.at[wid].at[pl.ds(off, seg)], dst_v)
            def chunk_body(j, carry):
                pltpu.async_copy(y_hbm.at[src_v.at[j]], rows_a, sem_a).wait()
                pltpu.sync_copy(rows_a, acc.at[dst_v.at[j]], add=True)
                return carry
            lax.fori_loop(0, seg, chunk_body, 0)
        plsc.subcore_barrier()
        pltpu.sync_copy(acc.at[pl.ds(base, rstripe)],
                        out_hbm.at[c].at[pl.ds(base, rstripe)])

    return pl.kernel(
        body,
        out_type=jax.ShapeDtypeStruct((_NC, acc_rows, d), jnp.float32),
        mesh=plsc.VectorSubcoreMesh(core_axis_name="c", subcore_axis_name="s"),
        scratch_types=[
            pltpu.VMEM((seg, _CHUNK), jnp.int32),
            pltpu.VMEM((seg, _CHUNK), jnp.int32),
            pltpu.VMEM((_CHUNK, d), jnp.float32),
            pltpu.VMEM((_CHUNK, d), jnp.float32),
            pltpu.VMEM_SHARED((acc_rows, d), jnp.float32),
            pltpu.SemaphoreType.DMA,
            pltpu.SemaphoreType.DMA,
        ],
    )


@functools.lru_cache(maxsize=None)
def _make_count_pass(n_nodes, nchunks):
    rstripe = _stripe_rows(n_nodes)
    acc_rows = rstripe * _NS

    def body(dst_hbm, out_hbm, dst_v, hist):
        c = lax.axis_index("c")
        s = lax.axis_index("s")
        wid = s * _NC + c
        pltpu.sync_copy(dst_hbm.at[wid], dst_v)

        def zero_body(i, carry):
            hist[pl.ds(i * 16, 16)] = jnp.zeros((16,), jnp.float32)
            return carry
        lax.fori_loop(0, acc_rows // 16, zero_body, 0)

        ones = jnp.ones((16,), jnp.float32)
        ipc = _CHUNK // 16

        def hist_body(i, carry):
            idx = dst_v[i // ipc, pl.ds((i % ipc) * 16, 16)]
            plsc.addupdate_scatter(hist, [idx], ones)
            return carry
        lax.fori_loop(0, nchunks * ipc, hist_body, 0)
        pltpu.sync_copy(hist, out_hbm.at[c].at[s])

    return pl.kernel(
        body,
        out_type=jax.ShapeDtypeStruct((_NC, _NS, acc_rows), jnp.float32),
        mesh=plsc.VectorSubcoreMesh(core_axis_name="c", subcore_axis_name="s"),
        scratch_types=[
            pltpu.VMEM((nchunks, _CHUNK), jnp.int32),
            pltpu.VMEM((acc_rows,), jnp.float32),
        ],
        compiler_params=pltpu.CompilerParams(needs_layout_passes=False),
    )


def _tcred(c3):
    nc, ns, acc_rows = c3.shape

    def body(c_ref, dj_ref):
        c3v = c_ref[...]
        cnt = jnp.sum(c3v[0] + c3v[1], axis=0)
        dinv = lax.rsqrt(cnt + 1.0)
        cntc = jnp.maximum(cnt, 1.0)
        z = jnp.zeros((acc_rows, 6), jnp.float32)
        dj_ref[...] = jnp.concatenate([dinv[:, None], cntc[:, None], z], axis=1)

    return pl.pallas_call(
        body,
        grid=(1,),
        in_specs=[pl.BlockSpec((nc, ns, acc_rows), lambda i: (0, 0, 0))],
        out_specs=pl.BlockSpec((acc_rows, 8), lambda i: (0, 0)),
        out_shape=jax.ShapeDtypeStruct((acc_rows, 8), jnp.float32),
    )(c3)


def _tc1(x, W1, dj):
    n, d = x.shape

    def body(x_ref, w_ref, dj_ref, y_ref, xw_ref):
        dinv = dj_ref[...][:, 0]
        xw = jnp.dot(x_ref[...], w_ref[...], preferred_element_type=jnp.float32)
        xw_ref[...] = xw
        y_ref[...] = xw * dinv[:, None]

    return pl.pallas_call(
        body,
        grid=(n // _R,),
        in_specs=[
            pl.BlockSpec((_R, d), lambda i: (i, 0)),
            pl.BlockSpec((d, d), lambda i: (0, 0)),
            pl.BlockSpec((_R, 8), lambda i: (i, 0)),
        ],
        out_specs=[pl.BlockSpec((_R, d), lambda i: (i, 0))] * 2,
        out_shape=[jax.ShapeDtypeStruct((n, d), jnp.float32)] * 2,
    )(x, W1, dj)


def _tc2(s1p, xw, dj, b1, W2r):
    n, d = xw.shape

    def body(s_ref, xw_ref, dj_ref, b_ref, w_ref, h1_ref, hr_ref):
        dinv = dj_ref[...][:, 0]
        sarr = s_ref[...]
        pre = ((sarr[0] + sarr[1]) * dinv[:, None]
               + xw_ref[...] * (dinv * dinv)[:, None] + b_ref[...])
        h1 = jnp.maximum(pre, 0.0)
        h1_ref[...] = h1
        hr_ref[...] = jnp.dot(h1, w_ref[...], preferred_element_type=jnp.float32)

    return pl.pallas_call(
        body,
        grid=(n // _R,),
        in_specs=[
            pl.BlockSpec((2, _R, d), lambda i: (0, i, 0)),
            pl.BlockSpec((_R, d), lambda i: (i, 0)),
            pl.BlockSpec((_R, 8), lambda i: (i, 0)),
            pl.BlockSpec((1, d), lambda i: (0, 0)),
            pl.BlockSpec((d, d), lambda i: (0, 0)),
        ],
        out_specs=[pl.BlockSpec((_R, d), lambda i: (i, 0))] * 2,
        out_shape=[jax.ShapeDtypeStruct((n, d), jnp.float32)] * 2,
    )(s1p, xw, dj, b1, W2r)


def _tc3(s2p, hr, dj, b2, W2l, W3):
    n, d = hr.shape

    def body(s_ref, hr_ref, dj_ref, b_ref, wl_ref, w3_ref, y3_ref, hw_ref):
        djv = dj_ref[...]
        dinv = djv[:, 0]
        cntc = djv[:, 1]
        sarr = s_ref[...]
        mean = (sarr[0] + sarr[1]) / cntc[:, None]
        h2 = jnp.maximum(
            jnp.dot(mean, wl_ref[...], preferred_element_type=jnp.float32)
            + hr_ref[...] + b_ref[...], 0.0)
        hw = jnp.dot(h2, w3_ref[...], preferred_element_type=jnp.float32)
        hw_ref[...] = hw
        y3_ref[...] = hw * dinv[:, None]

    return pl.pallas_call(
        body,
        grid=(n // _R,),
        in_specs=[
            pl.BlockSpec((2, _R, d), lambda i: (0, i, 0)),
            pl.BlockSpec((_R, d), lambda i: (i, 0)),
            pl.BlockSpec((_R, 8), lambda i: (i, 0)),
            pl.BlockSpec((1, d), lambda i: (0, 0)),
            pl.BlockSpec((d, d), lambda i: (0, 0)),
            pl.BlockSpec((d, d), lambda i: (0, 0)),
        ],
        out_specs=[pl.BlockSpec((_R, d), lambda i: (i, 0))] * 2,
        out_shape=[jax.ShapeDtypeStruct((n, d), jnp.float32)] * 2,
    )(s2p, hr, dj, b2, W2l, W3)


def _tc4(s3p, hw, dj, b3):
    n, d = hw.shape

    def body(s_ref, hw_ref, dj_ref, b_ref, o_ref):
        dinv = dj_ref[...][:, 0]
        sarr = s_ref[...]
        o_ref[...] = ((sarr[0] + sarr[1]) * dinv[:, None]
                      + hw_ref[...] * (dinv * dinv)[:, None] + b_ref[...])

    return pl.pallas_call(
        body,
        grid=(n // _R,),
        in_specs=[
            pl.BlockSpec((2, _R, d), lambda i: (0, i, 0)),
            pl.BlockSpec((_R, d), lambda i: (i, 0)),
            pl.BlockSpec((_R, 8), lambda i: (i, 0)),
            pl.BlockSpec((1, d), lambda i: (0, 0)),
        ],
        out_specs=pl.BlockSpec((_R, d), lambda i: (i, 0)),
        out_shape=jax.ShapeDtypeStruct((n, d), jnp.float32),
    )(s3p, hw, dj, b3)


def kernel(x, edge_index, W1, b1, W2l, W2r, b2, W3, b3):
    n, d = x.shape
    e = edge_index.shape[1]
    src = edge_index[0].astype(jnp.int32)
    dst = edge_index[1].astype(jnp.int32)

    nchunks = _ceil_div(_ceil_div(e, _NW * _CHUNK), _NSEG) * _NSEG
    e_pad = nchunks * _NW * _CHUNK
    srcp = jnp.concatenate(
        [src, jnp.zeros((e_pad - e,), jnp.int32)]).reshape(_NW, nchunks, _CHUNK)
    dstp = jnp.concatenate(
        [dst, jnp.full((e_pad - e,), n, jnp.int32)]).reshape(_NW, nchunks, _CHUNK)

    edge_pass = _make_edge_pass(n, d, nchunks)
    c3 = _make_count_pass(n, nchunks)(dstp)
    dj = _tcred(c3)

    y1, xw = _tc1(x, W1, dj)
    s1p = edge_pass(y1, srcp, dstp)
    h1, hr = _tc2(s1p, xw, dj, b1.reshape(1, d), W2r)
    s2p = edge_pass(h1, srcp, dstp)
    y3, hw = _tc3(s2p, hr, dj, b2.reshape(1, d), W2l, W3)
    s3p = edge_pass(y3, srcp, dstp)
    return _tc4(s3p, hw, dj, b3.reshape(1, d))


# speedup vs baseline: 1.5431x; 1.5431x over previous
import functools

import jax
import jax.numpy as jnp
from jax import lax
from jax.experimental import pallas as pl
from jax.experimental.pallas import tpu as pltpu
from jax.experimental.pallas import tpu_sc as plsc

_NC = 2
_NS = 16
_NW = _NC * _NS
_CHUNK = 128
_R = 1000


def _ceil_div(a, b):
    return -(-a // b)


def _stripe_rows(n_nodes):
    return _ceil_div(n_nodes + 1, _NS * _CHUNK) * _CHUNK


_NSEG = 1


@functools.lru_cache(maxsize=None)
def _make_edge_pass(n_nodes, d, nchunks):
    rstripe = _stripe_rows(n_nodes)
    acc_rows = rstripe * _NS
    seg = nchunks // _NSEG

    def body(y_hbm, src_hbm, dst_hbm, out_hbm, src_v, dst_v,
             rows_a, acc, sem_a):
        c = lax.axis_index("c")
        s = lax.axis_index("s")
        wid = s * _NC + c

        def zero_body(i, carry):
            rows_a[i // (d // 16), pl.ds((i % (d // 16)) * 16, 16)] = (
                jnp.zeros((16,), jnp.float32))
            return carry
        lax.fori_loop(0, _CHUNK * (d // 16), zero_body, 0)

        base = s * rstripe
        nfull = rstripe // _CHUNK
        rem = rstripe - nfull * _CHUNK
        for k in range(nfull):
            pltpu.sync_copy(rows_a, acc.at[pl.ds(base + k * _CHUNK, _CHUNK)])
        if rem:
            pltpu.sync_copy(rows_a.at[pl.ds(0, rem)],
                            acc.at[pl.ds(base + nfull * _CHUNK, rem)])

        plsc.subcore_barrier()

        for ph in range(_NSEG):
            off = ph * seg
            pltpu.sync_copy(src_hbm.at[wid].at[pl.ds(off, seg)], src_v)
            pltpu.sync_copy(dst_hbm.at[wid].at[pl.ds(off, seg)], dst_v)
            def chunk_body(j, carry):
                pltpu.async_copy(y_hbm.at[src_v.at[j]], rows_a, sem_a).wait()
                pltpu.sync_copy(rows_a, acc.at[dst_v.at[j]], add=True)
                return carry
            lax.fori_loop(0, seg, chunk_body, 0)
        plsc.subcore_barrier()
        pltpu.sync_copy(acc.at[pl.ds(base, rstripe)],
                        out_hbm.at[c].at[pl.ds(base, rstripe)])

    return pl.kernel(
        body,
        out_type=jax.ShapeDtypeStruct((_NC, acc_rows, d), jnp.float32),
        mesh=plsc.VectorSubcoreMesh(core_axis_name="c", subcore_axis_name="s"),
        scratch_types=[
            pltpu.VMEM((seg, _CHUNK), jnp.int32),
            pltpu.VMEM((seg, _CHUNK), jnp.int32),
            pltpu.VMEM((_CHUNK, d), jnp.float32),
            pltpu.VMEM_SHARED((acc_rows, d), jnp.float32),
            pltpu.SemaphoreType.DMA,
        ],
    )


@functools.lru_cache(maxsize=None)
def _make_count_pass(n_nodes, nchunks):
    rstripe = _stripe_rows(n_nodes)
    acc_rows = rstripe * _NS

    def body(dst_hbm, out_hbm, dst_v, hist):
        c = lax.axis_index("c")
        s = lax.axis_index("s")
        wid = s * _NC + c
        pltpu.sync_copy(dst_hbm.at[wid], dst_v)

        def zero_body(i, carry):
            hist[pl.ds(i * 16, 16)] = jnp.zeros((16,), jnp.float32)
            return carry
        lax.fori_loop(0, acc_rows // 16, zero_body, 0)

        ones = jnp.ones((16,), jnp.float32)
        ipc = _CHUNK // 16

        def hist_body(i, carry):
            idx = dst_v[i // ipc, pl.ds((i % ipc) * 16, 16)]
            plsc.addupdate_scatter(hist, [idx], ones)
            return carry
        lax.fori_loop(0, nchunks * ipc, hist_body, 0)
        pltpu.sync_copy(hist, out_hbm.at[c].at[s])

    return pl.kernel(
        body,
        out_type=jax.ShapeDtypeStruct((_NC, _NS, acc_rows), jnp.float32),
        mesh=plsc.VectorSubcoreMesh(core_axis_name="c", subcore_axis_name="s"),
        scratch_types=[
            pltpu.VMEM((nchunks, _CHUNK), jnp.int32),
            pltpu.VMEM((acc_rows,), jnp.float32),
        ],
        compiler_params=pltpu.CompilerParams(needs_layout_passes=False),
    )


def _tcred(c3):
    nc, ns, acc_rows = c3.shape

    def body(c_ref, dj_ref):
        c3v = c_ref[...]
        cnt = jnp.sum(c3v[0] + c3v[1], axis=0)
        dinv = lax.rsqrt(cnt + 1.0)
        cntc = jnp.maximum(cnt, 1.0)
        z = jnp.zeros((acc_rows, 6), jnp.float32)
        dj_ref[...] = jnp.concatenate([dinv[:, None], cntc[:, None], z], axis=1)

    return pl.pallas_call(
        body,
        grid=(1,),
        in_specs=[pl.BlockSpec((nc, ns, acc_rows), lambda i: (0, 0, 0))],
        out_specs=pl.BlockSpec((acc_rows, 8), lambda i: (0, 0)),
        out_shape=jax.ShapeDtypeStruct((acc_rows, 8), jnp.float32),
    )(c3)


def _tc1(x, W1, dj):
    n, d = x.shape

    def body(x_ref, w_ref, dj_ref, y_ref, xw_ref):
        dinv = dj_ref[...][:, 0]
        xw = jnp.dot(x_ref[...], w_ref[...], preferred_element_type=jnp.float32)
        xw_ref[...] = xw
        y_ref[...] = xw * dinv[:, None]

    return pl.pallas_call(
        body,
        grid=(n // _R,),
        in_specs=[
            pl.BlockSpec((_R, d), lambda i: (i, 0)),
            pl.BlockSpec((d, d), lambda i: (0, 0)),
            pl.BlockSpec((_R, 8), lambda i: (i, 0)),
        ],
        out_specs=[pl.BlockSpec((_R, d), lambda i: (i, 0))] * 2,
        out_shape=[jax.ShapeDtypeStruct((n, d), jnp.float32)] * 2,
    )(x, W1, dj)


def _tc2(s1p, xw, dj, b1, W2r):
    n, d = xw.shape

    def body(s_ref, xw_ref, dj_ref, b_ref, w_ref, h1_ref, hr_ref):
        dinv = dj_ref[...][:, 0]
        sarr = s_ref[...]
        pre = ((sarr[0] + sarr[1]) * dinv[:, None]
               + xw_ref[...] * (dinv * dinv)[:, None] + b_ref[...])
        h1 = jnp.maximum(pre, 0.0)
        h1_ref[...] = h1
        hr_ref[...] = jnp.dot(h1, w_ref[...], preferred_element_type=jnp.float32)

    return pl.pallas_call(
        body,
        grid=(n // _R,),
        in_specs=[
            pl.BlockSpec((2, _R, d), lambda i: (0, i, 0)),
            pl.BlockSpec((_R, d), lambda i: (i, 0)),
            pl.BlockSpec((_R, 8), lambda i: (i, 0)),
            pl.BlockSpec((1, d), lambda i: (0, 0)),
            pl.BlockSpec((d, d), lambda i: (0, 0)),
        ],
        out_specs=[pl.BlockSpec((_R, d), lambda i: (i, 0))] * 2,
        out_shape=[jax.ShapeDtypeStruct((n, d), jnp.float32)] * 2,
    )(s1p, xw, dj, b1, W2r)


def _tc3(s2p, hr, dj, b2, W2l, W3):
    n, d = hr.shape

    def body(s_ref, hr_ref, dj_ref, b_ref, wl_ref, w3_ref, y3_ref, hw_ref):
        djv = dj_ref[...]
        dinv = djv[:, 0]
        cntc = djv[:, 1]
        sarr = s_ref[...]
        mean = (sarr[0] + sarr[1]) / cntc[:, None]
        h2 = jnp.maximum(
            jnp.dot(mean, wl_ref[...], preferred_element_type=jnp.float32)
            + hr_ref[...] + b_ref[...], 0.0)
        hw = jnp.dot(h2, w3_ref[...], preferred_element_type=jnp.float32)
        hw_ref[...] = hw
        y3_ref[...] = hw * dinv[:, None]

    return pl.pallas_call(
        body,
        grid=(n // _R,),
        in_specs=[
            pl.BlockSpec((2, _R, d), lambda i: (0, i, 0)),
            pl.BlockSpec((_R, d), lambda i: (i, 0)),
            pl.BlockSpec((_R, 8), lambda i: (i, 0)),
            pl.BlockSpec((1, d), lambda i: (0, 0)),
            pl.BlockSpec((d, d), lambda i: (0, 0)),
            pl.BlockSpec((d, d), lambda i: (0, 0)),
        ],
        out_specs=[pl.BlockSpec((_R, d), lambda i: (i, 0))] * 2,
        out_shape=[jax.ShapeDtypeStruct((n, d), jnp.float32)] * 2,
    )(s2p, hr, dj, b2, W2l, W3)


def _tc4(s3p, hw, dj, b3):
    n, d = hw.shape

    def body(s_ref, hw_ref, dj_ref, b_ref, o_ref):
        dinv = dj_ref[...][:, 0]
        sarr = s_ref[...]
        o_ref[...] = ((sarr[0] + sarr[1]) * dinv[:, None]
                      + hw_ref[...] * (dinv * dinv)[:, None] + b_ref[...])

    return pl.pallas_call(
        body,
        grid=(n // _R,),
        in_specs=[
            pl.BlockSpec((2, _R, d), lambda i: (0, i, 0)),
            pl.BlockSpec((_R, d), lambda i: (i, 0)),
            pl.BlockSpec((_R, 8), lambda i: (i, 0)),
            pl.BlockSpec((1, d), lambda i: (0, 0)),
        ],
        out_specs=pl.BlockSpec((_R, d), lambda i: (i, 0)),
        out_shape=jax.ShapeDtypeStruct((n, d), jnp.float32),
    )(s3p, hw, dj, b3)


def kernel(x, edge_index, W1, b1, W2l, W2r, b2, W3, b3):
    n, d = x.shape
    e = edge_index.shape[1]
    src = edge_index[0].astype(jnp.int32)
    dst = edge_index[1].astype(jnp.int32)

    nchunks = _ceil_div(_ceil_div(e, _NW * _CHUNK), _NSEG) * _NSEG
    e_pad = nchunks * _NW * _CHUNK
    srcp = jnp.concatenate(
        [src, jnp.zeros((e_pad - e,), jnp.int32)]).reshape(_NW, nchunks, _CHUNK)
    dstp = jnp.concatenate(
        [dst, jnp.full((e_pad - e,), n, jnp.int32)]).reshape(_NW, nchunks, _CHUNK)

    edge_pass = _make_edge_pass(n, d, nchunks)
    c3 = _make_count_pass(n, nchunks)(dstp)
    dj = _tcred(c3)

    y1, xw = _tc1(x, W1, dj)
    s1p = edge_pass(y1, srcp, dstp)
    h1, hr = _tc2(s1p, xw, dj, b1.reshape(1, d), W2r)
    s2p = edge_pass(h1, srcp, dstp)
    y3, hw = _tc3(s2p, hr, dj, b2.reshape(1, d), W2l, W3)
    s3p = edge_pass(y3, srcp, dstp)
    return _tc4(s3p, hw, dj, b3.reshape(1, d))
